# Initial kernel scaffold; baseline (speedup 1.0000x reference)
import jax
import jax.numpy as jnp
from jax.experimental import pallas as pl


def kernel(x):
    raise NotImplementedError("write your pallas kernel here")


# speedup vs baseline: 1.4625x; 1.4625x over previous
import functools

import jax
import jax.numpy as jnp
import numpy as np
from jax import lax
from jax.experimental import pallas as pl
from jax.experimental.pallas import tpu as pltpu
from jax.experimental.pallas import tpu_sc as plsc

_B, _N, _D = 64, 576, 768
_KEEP = 288
_NW = 32
_ROWS_PER_W = (_B * _KEEP) // _NW
_CHUNK = 72
_NCHUNK = _ROWS_PER_W // _CHUNK


def _keep_row_indices():
    noise_key = jax.random.fold_in(jax.random.key(0), 1)
    noise = jax.random.normal(noise_key, (_B, _N), dtype=jnp.float32)
    _, keep = jax.lax.top_k(noise, _KEEP)
    gidx = keep.astype(jnp.int32) + (
        jnp.arange(_B, dtype=jnp.int32) * _N)[:, None]
    return gidx.reshape(_NW, _NCHUNK, _CHUNK)


@functools.partial(
    pl.kernel,
    mesh=plsc.VectorSubcoreMesh(core_axis_name="c", subcore_axis_name="s"),
    out_type=jax.ShapeDtypeStruct((_B * _KEEP, _D), jnp.float32),
    scratch_types=[
        pltpu.VMEM((_NCHUNK, _CHUNK), jnp.int32),
        pltpu.VMEM((_CHUNK, _D), jnp.float32),
        pltpu.VMEM((_CHUNK, _D), jnp.float32),
        pltpu.SemaphoreType.DMA,
        pltpu.SemaphoreType.DMA,
        pltpu.SemaphoreType.DMA,
        pltpu.SemaphoreType.DMA,
    ],
)
def _sc_gather(x_hbm, idx_hbm, out_hbm, idx_v, buf0, buf1,
               gsem0, gsem1, osem0, osem1):
    wid = lax.axis_index("s") * 2 + lax.axis_index("c")
    base = wid * _ROWS_PER_W
    bufs = (buf0, buf1)
    gsems = (gsem0, gsem1)
    osems = (osem0, osem1)

    pltpu.sync_copy(idx_hbm.at[wid], idx_v)

    gathers = [None] * _NCHUNK
    outs = [None] * _NCHUNK
    gathers[0] = pltpu.async_copy(x_hbm.at[idx_v.at[0]], bufs[0], gsems[0])
    for c in range(_NCHUNK):
        p = c & 1
        gathers[c].wait()
        if c + 1 < _NCHUNK:
            np_ = (c + 1) & 1
            if c - 1 >= 0:
                outs[c - 1].wait()
            gathers[c + 1] = pltpu.async_copy(
                x_hbm.at[idx_v.at[c + 1]], bufs[np_], gsems[np_])
        outs[c] = pltpu.async_copy(
            bufs[p], out_hbm.at[pl.ds(base + c * _CHUNK, _CHUNK)], osems[p])
    outs[_NCHUNK - 2].wait()
    outs[_NCHUNK - 1].wait()


def kernel(x):
    b, n, d = x.shape
    gidx = _keep_row_indices()
    out = _sc_gather(x.reshape(b * n, d), gidx)
    return out.reshape(_B, _KEEP, _D)


# speedup vs baseline: 1.9216x; 1.3139x over previous
import functools

import jax
import jax.numpy as jnp
import numpy as np
from jax import lax
from jax.experimental import pallas as pl
from jax.experimental.pallas import tpu as pltpu
from jax.experimental.pallas import tpu_sc as plsc

_B, _N, _D = 64, 576, 768
_KEEP = 288
_NW = 32
_ROWS_PER_W = (_B * _KEEP) // _NW
_CHUNK = 72
_NCHUNK = _ROWS_PER_W // _CHUNK


def _keep_row_indices_expr():
    noise_key = jax.random.fold_in(jax.random.key(0), 1)
    noise = jax.random.normal(noise_key, (_B, _N), dtype=jnp.float32)
    _, keep = jax.lax.top_k(noise, _KEEP)
    gidx = keep.astype(jnp.int32) + (
        jnp.arange(_B, dtype=jnp.int32) * _N)[:, None]
    return gidx.reshape(_NW, _NCHUNK, _CHUNK)


_GIDX_CACHE = []


def _keep_row_indices():
    if _GIDX_CACHE:
        return jnp.asarray(_GIDX_CACHE[0])
    try:
        with jax.ensure_compile_time_eval():
            gidx = np.asarray(_keep_row_indices_expr())
        _GIDX_CACHE.append(gidx)
        return jnp.asarray(gidx)
    except Exception:
        return _keep_row_indices_expr()


@functools.partial(
    pl.kernel,
    mesh=plsc.VectorSubcoreMesh(core_axis_name="c", subcore_axis_name="s"),
    out_type=jax.ShapeDtypeStruct((_B * _KEEP, _D), jnp.float32),
    scratch_types=[
        pltpu.VMEM((_NCHUNK, _CHUNK), jnp.int32),
        pltpu.VMEM((_CHUNK, _D), jnp.float32),
        pltpu.VMEM((_CHUNK, _D), jnp.float32),
        pltpu.SemaphoreType.DMA,
        pltpu.SemaphoreType.DMA,
        pltpu.SemaphoreType.DMA,
        pltpu.SemaphoreType.DMA,
    ],
)
def _sc_gather(x_hbm, idx_hbm, out_hbm, idx_v, buf0, buf1,
               gsem0, gsem1, osem0, osem1):
    wid = lax.axis_index("s") * 2 + lax.axis_index("c")
    base = wid * _ROWS_PER_W
    bufs = (buf0, buf1)
    gsems = (gsem0, gsem1)
    osems = (osem0, osem1)

    pltpu.sync_copy(idx_hbm.at[wid], idx_v)

    gathers = [None] * _NCHUNK
    outs = [None] * _NCHUNK
    gathers[0] = pltpu.async_copy(x_hbm.at[idx_v.at[0]], bufs[0], gsems[0])
    for c in range(_NCHUNK):
        p = c & 1
        gathers[c].wait()
        if c + 1 < _NCHUNK:
            np_ = (c + 1) & 1
            if c - 1 >= 0:
                outs[c - 1].wait()
            gathers[c + 1] = pltpu.async_copy(
                x_hbm.at[idx_v.at[c + 1]], bufs[np_], gsems[np_])
        outs[c] = pltpu.async_copy(
            bufs[p], out_hbm.at[pl.ds(base + c * _CHUNK, _CHUNK)], osems[p])
    outs[_NCHUNK - 2].wait()
    outs[_NCHUNK - 1].wait()


def kernel(x):
    b, n, d = x.shape
    gidx = _keep_row_indices()
    out = _sc_gather(x.reshape(b * n, d), gidx)
    return out.reshape(_B, _KEEP, _D)


# speedup vs baseline: 1.9756x; 1.0281x over previous
import functools

import jax
import jax.numpy as jnp
import numpy as np
from jax import lax
from jax.experimental import pallas as pl
from jax.experimental.pallas import tpu as pltpu
from jax.experimental.pallas import tpu_sc as plsc

_B, _N, _D = 64, 576, 768
_KEEP = 288
_NW = 32
_ROWS_PER_W = (_B * _KEEP) // _NW
_CHUNK = 48
_NCHUNK = _ROWS_PER_W // _CHUNK
_NBUF = 3


def _keep_row_indices_expr():
    noise_key = jax.random.fold_in(jax.random.key(0), 1)
    noise = jax.random.normal(noise_key, (_B, _N), dtype=jnp.float32)
    _, keep = jax.lax.top_k(noise, _KEEP)
    gidx = keep.astype(jnp.int32) + (
        jnp.arange(_B, dtype=jnp.int32) * _N)[:, None]
    return gidx.reshape(_NW, _NCHUNK, _CHUNK)


_GIDX_CACHE = []


def _keep_row_indices():
    if _GIDX_CACHE:
        return jnp.asarray(_GIDX_CACHE[0])
    try:
        with jax.ensure_compile_time_eval():
            gidx = np.asarray(_keep_row_indices_expr())
        _GIDX_CACHE.append(gidx)
        return jnp.asarray(gidx)
    except Exception:
        return _keep_row_indices_expr()


@functools.partial(
    pl.kernel,
    mesh=plsc.VectorSubcoreMesh(core_axis_name="c", subcore_axis_name="s"),
    out_type=jax.ShapeDtypeStruct((_B * _KEEP, _D), jnp.float32),
    scratch_types=(
        [pltpu.VMEM((_NCHUNK, _CHUNK), jnp.int32)]
        + [pltpu.VMEM((_CHUNK, _D), jnp.float32)] * _NBUF
        + [pltpu.SemaphoreType.DMA] * (2 * _NBUF)
    ),
)
def _sc_gather(x_hbm, idx_hbm, out_hbm, idx_v, *bufs_sems):
    bufs = bufs_sems[:_NBUF]
    gsems = bufs_sems[_NBUF:2 * _NBUF]
    osems = bufs_sems[2 * _NBUF:]
    wid = lax.axis_index("s") * 2 + lax.axis_index("c")
    base = wid * _ROWS_PER_W

    pltpu.sync_copy(idx_hbm.at[wid], idx_v)

    def gather(c):
        p = c % _NBUF
        return pltpu.async_copy(x_hbm.at[idx_v.at[c]], bufs[p], gsems[p])

    gathers = [None] * _NCHUNK
    outs = [None] * _NCHUNK
    for c in range(min(_NBUF - 1, _NCHUNK)):
        gathers[c] = gather(c)
    for c in range(_NCHUNK):
        p = c % _NBUF
        gathers[c].wait()
        n = c + _NBUF - 1
        if n < _NCHUNK:
            if c - 1 >= 0:
                outs[c - 1].wait()
            gathers[n] = gather(n)
        outs[c] = pltpu.async_copy(
            bufs[p], out_hbm.at[pl.ds(base + c * _CHUNK, _CHUNK)], osems[p])
    for c in range(max(0, _NCHUNK - _NBUF), _NCHUNK):
        outs[c].wait()


def kernel(x):
    b, n, d = x.shape
    gidx = _keep_row_indices()
    out = _sc_gather(x.reshape(b * n, d), gidx)
    return out.reshape(_B, _KEEP, _D)


# speedup vs baseline: 1.9793x; 1.0019x over previous
import functools

import jax
import jax.numpy as jnp
import numpy as np
from jax import lax
from jax.experimental import pallas as pl
from jax.experimental.pallas import tpu as pltpu
from jax.experimental.pallas import tpu_sc as plsc

_B, _N, _D = 64, 576, 768
_KEEP = 288
_NW = 32
_ROWS_PER_W = (_B * _KEEP) // _NW
_CHUNK = 32
_NCHUNK = _ROWS_PER_W // _CHUNK
_NBUF = 4


def _keep_row_indices_expr():
    noise_key = jax.random.fold_in(jax.random.key(0), 1)
    noise = jax.random.normal(noise_key, (_B, _N), dtype=jnp.float32)
    _, keep = jax.lax.top_k(noise, _KEEP)
    gidx = keep.astype(jnp.int32) + (
        jnp.arange(_B, dtype=jnp.int32) * _N)[:, None]
    return gidx.reshape(_NW, _NCHUNK, _CHUNK)


_GIDX_CACHE = []


def _keep_row_indices():
    if _GIDX_CACHE:
        return jnp.asarray(_GIDX_CACHE[0])
    try:
        with jax.ensure_compile_time_eval():
            gidx = np.asarray(_keep_row_indices_expr())
        _GIDX_CACHE.append(gidx)
        return jnp.asarray(gidx)
    except Exception:
        return _keep_row_indices_expr()


@functools.partial(
    pl.kernel,
    mesh=plsc.VectorSubcoreMesh(core_axis_name="c", subcore_axis_name="s"),
    out_type=jax.ShapeDtypeStruct((_B * _KEEP, _D), jnp.float32),
    scratch_types=(
        [pltpu.VMEM((_NCHUNK, _CHUNK), jnp.int32)]
        + [pltpu.VMEM((_CHUNK, _D), jnp.float32)] * _NBUF
        + [pltpu.SemaphoreType.DMA] * (2 * _NBUF)
    ),
)
def _sc_gather(x_hbm, idx_hbm, out_hbm, idx_v, *bufs_sems):
    bufs = bufs_sems[:_NBUF]
    gsems = bufs_sems[_NBUF:2 * _NBUF]
    osems = bufs_sems[2 * _NBUF:]
    wid = lax.axis_index("s") * 2 + lax.axis_index("c")
    base = wid * _ROWS_PER_W

    pltpu.sync_copy(idx_hbm.at[wid], idx_v)

    def gather(c):
        p = c % _NBUF
        return pltpu.async_copy(x_hbm.at[idx_v.at[c]], bufs[p], gsems[p])

    gathers = [None] * _NCHUNK
    outs = [None] * _NCHUNK
    for c in range(min(_NBUF - 1, _NCHUNK)):
        gathers[c] = gather(c)
    for c in range(_NCHUNK):
        p = c % _NBUF
        gathers[c].wait()
        n = c + _NBUF - 1
        if n < _NCHUNK:
            if c - 1 >= 0:
                outs[c - 1].wait()
            gathers[n] = gather(n)
        outs[c] = pltpu.async_copy(
            bufs[p], out_hbm.at[pl.ds(base + c * _CHUNK, _CHUNK)], osems[p])
    for c in range(max(0, _NCHUNK - _NBUF), _NCHUNK):
        outs[c].wait()


def kernel(x):
    b, n, d = x.shape
    gidx = _keep_row_indices()
    out = _sc_gather(x.reshape(b * n, d), gidx)
    return out.reshape(_B, _KEEP, _D)
